# baseline (speedup 1.0000x reference)
import functools

import jax
import jax.numpy as jnp
import numpy as np
from jax import lax
from jax.experimental import pallas as pl
from jax.experimental.pallas import tpu as pltpu
from jax.experimental.pallas import tpu_sc as plsc

STEP = 16000
N_FRAMES = 128
CHANNELS = 2
LENGTH = N_FRAMES * STEP

_PERM = [
    19, 76, 118, 54, 90, 30, 7, 96, 121, 115, 6, 35, 23, 58, 16, 21,
    77, 94, 116, 61, 38, 3, 105, 81, 26, 32, 64, 37, 56, 51, 2, 122,
    63, 52, 20, 89, 95, 44, 47, 123, 79, 84, 50, 78, 72, 83, 42, 62,
    69, 53, 0, 8, 109, 22, 13, 29, 99, 110, 34, 70, 18, 103, 86, 75,
    91, 111, 24, 113, 1, 65, 48, 5, 45, 49, 33, 74, 55, 60, 119, 57,
    124, 27, 112, 10, 93, 68, 15, 73, 40, 67, 88, 102, 107, 66, 80, 100,
    120, 71, 17, 59, 98, 108, 114, 36, 125, 101, 92, 28, 46, 9, 104, 117,
    4, 12, 87, 85, 14, 82, 31, 106, 127, 126, 97, 41, 25, 43, 39, 11,
]

_NC = 2
_NS = 16
_NW = _NC * _NS
_FPW = N_FRAMES // _NW

_SRC_OFF = np.zeros((_NW, 128), dtype=np.int32)
for _w in range(_NW):
    for _j in range(_FPW):
        _SRC_OFF[_w, _j] = _PERM[_w * _FPW + _j] * STEP

_mesh = plsc.VectorSubcoreMesh(core_axis_name="c", subcore_axis_name="s")


@functools.partial(
    pl.kernel,
    mesh=_mesh,
    out_type=jax.ShapeDtypeStruct((CHANNELS, LENGTH), jnp.float32),
    scratch_types=[
        pltpu.VMEM((128,), jnp.int32),
        pltpu.VMEM((_FPW * CHANNELS, STEP), jnp.float32),
        pltpu.SemaphoreType.DMA((_FPW,)),
        pltpu.SemaphoreType.DMA,
    ],
)
def _shuffle(src_hbm, offs_hbm, out_hbm, offs_v, slabs_v, gsem, ssem):
    wid = lax.axis_index("s") * _NC + lax.axis_index("c")
    pltpu.sync_copy(offs_hbm.at[wid], offs_v)
    offs = offs_v[pl.ds(0, 16)]
    frame_base = wid * _FPW

    gathers = []
    for j in range(_FPW):
        off = pl.multiple_of(offs[j], STEP)
        gathers.append(
            pltpu.async_copy(
                src_hbm.at[:, pl.ds(off, STEP)],
                slabs_v.at[pl.ds(j * CHANNELS, CHANNELS)],
                gsem.at[j],
            )
        )
    scatters = []
    for j in range(_FPW):
        gathers[j].wait()
        doff = pl.multiple_of((frame_base + j) * STEP, STEP)
        scatters.append(
            pltpu.async_copy(
                slabs_v.at[pl.ds(j * CHANNELS, CHANNELS)],
                out_hbm.at[:, pl.ds(doff, STEP)],
                ssem,
            )
        )
    for s in scatters:
        s.wait()


def kernel(waveform):
    return _shuffle(waveform, jnp.asarray(_SRC_OFF))


# speedup vs baseline: 1.0037x; 1.0037x over previous
import functools

import jax
import jax.numpy as jnp
from jax import lax
from jax.experimental import pallas as pl
from jax.experimental.pallas import tpu as pltpu
from jax.experimental.pallas import tpu_sc as plsc

STEP = 16000
N_FRAMES = 128
CHANNELS = 2
LENGTH = N_FRAMES * STEP

_PERM = [
    19, 76, 118, 54, 90, 30, 7, 96, 121, 115, 6, 35, 23, 58, 16, 21,
    77, 94, 116, 61, 38, 3, 105, 81, 26, 32, 64, 37, 56, 51, 2, 122,
    63, 52, 20, 89, 95, 44, 47, 123, 79, 84, 50, 78, 72, 83, 42, 62,
    69, 53, 0, 8, 109, 22, 13, 29, 99, 110, 34, 70, 18, 103, 86, 75,
    91, 111, 24, 113, 1, 65, 48, 5, 45, 49, 33, 74, 55, 60, 119, 57,
    124, 27, 112, 10, 93, 68, 15, 73, 40, 67, 88, 102, 107, 66, 80, 100,
    120, 71, 17, 59, 98, 108, 114, 36, 125, 101, 92, 28, 46, 9, 104, 117,
    4, 12, 87, 85, 14, 82, 31, 106, 127, 126, 97, 41, 25, 43, 39, 11,
]

_NC = 2
_NS = 16
_NW = _NC * _NS
_FPW = N_FRAMES // _NW

_mesh = plsc.VectorSubcoreMesh(core_axis_name="c", subcore_axis_name="s")


@functools.partial(
    pl.kernel,
    mesh=_mesh,
    out_type=jax.ShapeDtypeStruct((CHANNELS, LENGTH), jnp.float32),
    scratch_types=[
        pltpu.SMEM((_FPW,), jnp.int32),
        pltpu.VMEM((_FPW * CHANNELS, STEP), jnp.float32),
        pltpu.SemaphoreType.DMA((_FPW,)),
        pltpu.SemaphoreType.DMA,
    ],
)
def _shuffle(src_hbm, out_hbm, offs_s, slabs_v, gsem, ssem):
    wid = lax.axis_index("s") * _NC + lax.axis_index("c")
    for w in range(_NW):

        @pl.when(wid == w)
        def _(w=w):
            for j in range(_FPW):
                offs_s[j] = _PERM[w * _FPW + j] * STEP

    frame_base = wid * _FPW

    gathers = []
    for j in range(_FPW):
        off = pl.multiple_of(offs_s[j], STEP)
        gathers.append(
            pltpu.async_copy(
                src_hbm.at[:, pl.ds(off, STEP)],
                slabs_v.at[pl.ds(j * CHANNELS, CHANNELS)],
                gsem.at[j],
            )
        )
    scatters = []
    for j in range(_FPW):
        gathers[j].wait()
        doff = pl.multiple_of((frame_base + j) * STEP, STEP)
        scatters.append(
            pltpu.async_copy(
                slabs_v.at[pl.ds(j * CHANNELS, CHANNELS)],
                out_hbm.at[:, pl.ds(doff, STEP)],
                ssem,
            )
        )
    for s in scatters:
        s.wait()


def kernel(waveform):
    return _shuffle(waveform)
